# baseline (speedup 1.0000x reference)
import functools

import jax
import jax.numpy as jnp
from jax import lax
from jax.experimental import pallas as pl
from jax.experimental.pallas import tpu as pltpu
from jax.experimental.pallas import tpu_sc as plsc

D = 1024
F = 2048
E = 8
K = 2
N = 4096
TM = 256
NPAD = N * K + E * TM
NTILES = NPAD // TM
EPAD = 128
NEG = -1e30
BIG = 1 << 29
NW = 32
TPW = N // NW
TPW16 = N // 16
CH = 32


def _router_body(xref, wgref, bgref, v1ref, v2ref, oh1ref, oh2ref):
    xt = xref[...]
    logits = jnp.dot(xt, wgref[...], preferred_element_type=jnp.float32)
    logits = logits + bgref[...]
    probs = jax.nn.softmax(logits, axis=-1)
    lane = lax.broadcasted_iota(jnp.int32, probs.shape, 1)
    m1 = jnp.max(probs, axis=-1, keepdims=True)
    am1 = jnp.min(jnp.where(probs == m1, lane, EPAD), axis=-1, keepdims=True)
    rest = jnp.where(lane == am1, NEG, probs)
    m2 = jnp.max(rest, axis=-1, keepdims=True)
    am2 = jnp.min(jnp.where(rest == m2, lane, EPAD), axis=-1, keepdims=True)
    v1ref[...] = jnp.broadcast_to(m1, (TM, 16))
    v2ref[...] = jnp.broadcast_to(m2, (TM, 16))
    oh1ref[...] = jnp.broadcast_to(am1, (TM, 16))
    oh2ref[...] = jnp.broadcast_to(am2, (TM, 16))


def _router(x_flat, Wg, bg):
    Wgp = jnp.zeros((D, EPAD), jnp.float32).at[:, :E].set(Wg)
    bgp = jnp.full((1, EPAD), NEG, jnp.float32).at[0, :E].set(bg)
    outs = pl.pallas_call(
        _router_body,
        grid=(N // TM,),
        in_specs=[
            pl.BlockSpec((TM, D), lambda t: (t, 0)),
            pl.BlockSpec((D, EPAD), lambda t: (0, 0)),
            pl.BlockSpec((1, EPAD), lambda t: (0, 0)),
        ],
        out_specs=[pl.BlockSpec((TM, 16), lambda t: (t, 0))] * 4,
        out_shape=[
            jax.ShapeDtypeStruct((N, 16), jnp.float32),
            jax.ShapeDtypeStruct((N, 16), jnp.float32),
            jax.ShapeDtypeStruct((N, 16), jnp.int32),
            jax.ShapeDtypeStruct((N, 16), jnp.int32),
        ],
    )(x_flat, Wgp, bgp)
    return outs


_GDN = lax.GatherDimensionNumbers(offset_dims=(), collapsed_slice_dims=(0,),
                                  start_index_map=(0,))


def _dyn_gather(arr, idx):
    return lax.gather(arr, idx[:, None], _GDN, (1,),
                      mode=lax.GatherScatterMode.PROMISE_IN_BOUNDS)


def _routing_body(oh1_hbm, oh2_hbm, pos1_hbm, pos2_hbm, toff_hbm,
                  oh1v, oh2v, posv, rdv, cntv, toffv, sharedv):
    cid = lax.axis_index("c")
    sid = lax.axis_index("s")
    base = sid * TPW16
    pltpu.sync_copy(oh1_hbm.at[pl.ds(base, TPW16), :], oh1v)
    pltpu.sync_copy(oh2_hbm.at[pl.ds(base, TPW16), :], oh2v)
    iota16 = lax.broadcasted_iota(jnp.int32, (16,), 0)
    zeros16 = jnp.zeros((16,), jnp.int32)

    def _cnt(j, c):
        c = jnp.where(iota16 == oh1v[j], c + 1, c)
        return jnp.where(iota16 == oh2v[j], c + 1, c)
    cnt = lax.fori_loop(0, TPW16, _cnt, zeros16)
    cntv[...] = cnt
    pltpu.sync_copy(cntv, sharedv.at[pl.ds(sid * 16, 16)])
    plsc.subcore_barrier()
    for w in range(16):
        pltpu.sync_copy(sharedv.at[pl.ds(w * 16, 16)], rdv.at[w])

    totals = zeros16
    mybase = zeros16
    for w in range(16):
        row = rdv[w]
        totals = totals + row
        gate = jnp.minimum(jnp.maximum(sid - w, 0), 1)
        mybase = mybase + row * gate
    aligned = ((totals + (TM - 1)) >> 8) << 8
    csum = aligned
    for dstep in (1, 2, 4, 8):
        shifted = _dyn_gather(csum, jnp.maximum(iota16 - dstep, 0))
        csum = jnp.where(iota16 >= dstep, csum + shifted, csum)
    offs = csum - aligned
    tile_starts = jnp.where(iota16 < E, offs >> 8,
                            jnp.full((16,), BIG, jnp.int32))
    nextpos = offs + mybase

    @pl.when((cid == 0) & (sid == 0))
    def _():
        toffv[...] = tile_starts
        pltpu.sync_copy(toffv, toff_hbm)

    def _mk_pass(ohv):
        def _chunk(cc, np_):
            packed = zeros16
            for j in range(16):
                es = ohv[cc * 16 + j]
                s_splat = _dyn_gather(np_, es)
                packed = jnp.where(iota16 == j, s_splat, packed)
                np_ = jnp.where(iota16 == es, np_ + 1, np_)
            posv[pl.ds(cc * 16, 16)] = packed
            return np_
        return _chunk

    nextpos = lax.fori_loop(0, TPW16 // 16, _mk_pass(oh1v), nextpos)

    @pl.when(cid == 0)
    def _():
        pltpu.sync_copy(posv, pos1_hbm.at[pl.ds(base, TPW16)])

    nextpos = lax.fori_loop(0, TPW16 // 16, _mk_pass(oh2v), nextpos)

    @pl.when(cid == 0)
    def _():
        pltpu.sync_copy(posv, pos2_hbm.at[pl.ds(base, TPW16)])


def _routing(oh1, oh2, interpret=False):
    fn = pl.kernel(
        _routing_body,
        interpret=interpret,
        out_type=[
            jax.ShapeDtypeStruct((N,), jnp.int32),
            jax.ShapeDtypeStruct((N,), jnp.int32),
            jax.ShapeDtypeStruct((16,), jnp.int32),
        ],
        mesh=plsc.VectorSubcoreMesh(core_axis_name="c", subcore_axis_name="s",
                                    num_cores=2, num_subcores=16),
        scratch_types=[
            pltpu.VMEM((TPW16, 16), jnp.int32),
            pltpu.VMEM((TPW16, 16), jnp.int32),
            pltpu.VMEM((TPW16,), jnp.int32),
            pltpu.VMEM((16, 16), jnp.int32),
            pltpu.VMEM((16,), jnp.int32),
            pltpu.VMEM((16,), jnp.int32),
            pltpu.VMEM_SHARED((256,), jnp.int32),
        ],
    )
    return fn(oh1, oh2)


def _scatter_body(x_hbm, pos1_hbm, pos2_hbm, xs_hbm, xbuf, idxv, sem):
    cid = lax.axis_index("c")
    sid = lax.axis_index("s")
    wid = sid * 2 + cid
    base = wid * TPW
    for c in range(TPW // CH):
        pltpu.sync_copy(x_hbm.at[pl.ds(base + c * CH, CH), :], xbuf)
        pltpu.sync_copy(pos1_hbm.at[pl.ds(base + c * CH, CH)], idxv)
        pltpu.async_copy(xbuf, xs_hbm.at[idxv], sem).wait()
        pltpu.sync_copy(pos2_hbm.at[pl.ds(base + c * CH, CH)], idxv)
        pltpu.async_copy(xbuf, xs_hbm.at[idxv], sem).wait()


def _dispatch(x_flat, pos1, pos2):
    fn = pl.kernel(
        _scatter_body,
        out_type=jax.ShapeDtypeStruct((NPAD, D), jnp.float32),
        mesh=plsc.VectorSubcoreMesh(core_axis_name="c", subcore_axis_name="s"),
        scratch_types=[
            pltpu.VMEM((CH, D), jnp.float32),
            pltpu.VMEM((CH,), jnp.int32),
            pltpu.SemaphoreType.DMA,
        ],
    )
    return fn(x_flat, pos1, pos2)


def _ffn_body(toff, xref, w1ref, b1ref, w2ref, b2ref, yref):
    xt = xref[...].astype(jnp.bfloat16)
    h = jax.nn.relu(jnp.dot(xt, w1ref[0], preferred_element_type=jnp.float32)
                    + b1ref[0])
    yref[...] = (jnp.dot(h.astype(jnp.bfloat16), w2ref[0],
                         preferred_element_type=jnp.float32) + b2ref[0])


def _tile_expert(t, toff):
    e = jnp.int32(0)
    for j in range(1, 16):
        e = e + (t >= toff[j]).astype(jnp.int32)
    return e


def _ffn(toff, xs, W1, b1, W2, b2):
    grid_spec = pltpu.PrefetchScalarGridSpec(
        num_scalar_prefetch=1,
        grid=(NTILES,),
        in_specs=[
            pl.BlockSpec((TM, D), lambda t, toff: (t, 0)),
            pl.BlockSpec((1, D, F), lambda t, toff: (_tile_expert(t, toff), 0, 0)),
            pl.BlockSpec((1, 1, F), lambda t, toff: (_tile_expert(t, toff), 0, 0)),
            pl.BlockSpec((1, F, D), lambda t, toff: (_tile_expert(t, toff), 0, 0)),
            pl.BlockSpec((1, 1, D), lambda t, toff: (_tile_expert(t, toff), 0, 0)),
        ],
        out_specs=pl.BlockSpec((TM, D), lambda t, toff: (t, 0)),
    )
    return pl.pallas_call(
        _ffn_body,
        grid_spec=grid_spec,
        out_shape=jax.ShapeDtypeStruct((NPAD, D), jnp.float32),
    )(toff, xs, W1.astype(jnp.bfloat16), b1.reshape(E, 1, F),
      W2.astype(jnp.bfloat16), b2.reshape(E, 1, D))


def _gather_body(y_hbm, pos1_hbm, pos2_hbm, ya_hbm, yb_hbm, buf, idxv, sem):
    cid = lax.axis_index("c")
    sid = lax.axis_index("s")
    wid = sid * 2 + cid
    base = wid * TPW
    for c in range(TPW // CH):
        pltpu.sync_copy(pos1_hbm.at[pl.ds(base + c * CH, CH)], idxv)
        pltpu.async_copy(y_hbm.at[idxv], buf, sem).wait()
        pltpu.sync_copy(buf, ya_hbm.at[pl.ds(base + c * CH, CH), :])
        pltpu.sync_copy(pos2_hbm.at[pl.ds(base + c * CH, CH)], idxv)
        pltpu.async_copy(y_hbm.at[idxv], buf, sem).wait()
        pltpu.sync_copy(buf, yb_hbm.at[pl.ds(base + c * CH, CH), :])


def _gather(y, pos1, pos2):
    fn = pl.kernel(
        _gather_body,
        out_type=[
            jax.ShapeDtypeStruct((N, D), jnp.float32),
            jax.ShapeDtypeStruct((N, D), jnp.float32),
        ],
        mesh=plsc.VectorSubcoreMesh(core_axis_name="c", subcore_axis_name="s"),
        scratch_types=[
            pltpu.VMEM((CH, D), jnp.float32),
            pltpu.VMEM((CH,), jnp.int32),
            pltpu.SemaphoreType.DMA,
        ],
    )
    return fn(y, pos1, pos2)


def _combine_body(yaref, ybref, v1ref, v2ref, oref):
    oref[...] = (yaref[...] * v1ref[...][:, :1]
                 + ybref[...] * v2ref[...][:, :1])


def _combine(ya, yb, v1b, v2b):
    return pl.pallas_call(
        _combine_body,
        grid=(N // TM,),
        in_specs=[
            pl.BlockSpec((TM, D), lambda t: (t, 0)),
            pl.BlockSpec((TM, D), lambda t: (t, 0)),
            pl.BlockSpec((TM, 16), lambda t: (t, 0)),
            pl.BlockSpec((TM, 16), lambda t: (t, 0)),
        ],
        out_specs=pl.BlockSpec((TM, D), lambda t: (t, 0)),
        out_shape=jax.ShapeDtypeStruct((N, D), jnp.float32),
    )(ya, yb, v1b, v2b)


def kernel(x, Wg, bg, W1, b1, W2, b2):
    B, T, _ = x.shape
    x_flat = x.reshape(N, D)
    v1b, v2b, oh1, oh2 = _router(x_flat, Wg, bg)
    pos1, pos2, toff = _routing(oh1, oh2)
    xs = _dispatch(x_flat, pos1, pos2)
    y = _ffn(toff, xs, W1, b1, W2, b2)
    ya, yb = _gather(y, pos1, pos2)
    out = _combine(ya, yb, v1b, v2b)
    return out.reshape(B, T, D)


# speedup vs baseline: 1.1725x; 1.1725x over previous
import functools

import jax
import jax.numpy as jnp
from jax import lax
from jax.experimental import pallas as pl
from jax.experimental.pallas import tpu as pltpu
from jax.experimental.pallas import tpu_sc as plsc

D = 1024
F = 2048
E = 8
K = 2
N = 4096
TM = 256
NPAD = N * K + E * TM
NTILES = NPAD // TM
EPAD = 128
NEG = -1e30
BIG = 1 << 29
NW = 32
TPW = N // NW
TPW16 = N // 16
CH = 32


def _router_body(xref, wgref, bgref, v1ref, v2ref, oh1ref, oh2ref):
    xt = xref[...]
    logits = jnp.dot(xt, wgref[...], preferred_element_type=jnp.float32)
    logits = logits + bgref[...]
    probs = jax.nn.softmax(logits, axis=-1)
    lane = lax.broadcasted_iota(jnp.int32, probs.shape, 1)
    m1 = jnp.max(probs, axis=-1, keepdims=True)
    am1 = jnp.min(jnp.where(probs == m1, lane, EPAD), axis=-1, keepdims=True)
    rest = jnp.where(lane == am1, NEG, probs)
    m2 = jnp.max(rest, axis=-1, keepdims=True)
    am2 = jnp.min(jnp.where(rest == m2, lane, EPAD), axis=-1, keepdims=True)
    v1ref[...] = jnp.broadcast_to(m1, (TM, 16))
    v2ref[...] = jnp.broadcast_to(m2, (TM, 16))
    oh1ref[...] = jnp.broadcast_to(am1, (TM, 16))
    oh2ref[...] = jnp.broadcast_to(am2, (TM, 16))


def _router(x_flat, Wg, bg):
    Wgp = jnp.zeros((D, EPAD), jnp.float32).at[:, :E].set(Wg)
    bgp = jnp.full((1, EPAD), NEG, jnp.float32).at[0, :E].set(bg)
    outs = pl.pallas_call(
        _router_body,
        grid=(N // TM,),
        in_specs=[
            pl.BlockSpec((TM, D), lambda t: (t, 0)),
            pl.BlockSpec((D, EPAD), lambda t: (0, 0)),
            pl.BlockSpec((1, EPAD), lambda t: (0, 0)),
        ],
        out_specs=[pl.BlockSpec((TM, 16), lambda t: (t, 0))] * 4,
        out_shape=[
            jax.ShapeDtypeStruct((N, 16), jnp.float32),
            jax.ShapeDtypeStruct((N, 16), jnp.float32),
            jax.ShapeDtypeStruct((N, 16), jnp.int32),
            jax.ShapeDtypeStruct((N, 16), jnp.int32),
        ],
    )(x_flat, Wgp, bgp)
    return outs


_GDN = lax.GatherDimensionNumbers(offset_dims=(), collapsed_slice_dims=(0,),
                                  start_index_map=(0,))


def _dyn_gather(arr, idx):
    return lax.gather(arr, idx[:, None], _GDN, (1,),
                      mode=lax.GatherScatterMode.PROMISE_IN_BOUNDS)


def _routing_body(oh1_hbm, oh2_hbm, x_hbm, pos1_hbm, pos2_hbm, toff_hbm,
                  xs_hbm, oh1v, oh2v, pos1v, pos2v, rdv, cntv, toffv, xbuf,
                  idxv, sharedv, sem):
    cid = lax.axis_index("c")
    sid = lax.axis_index("s")
    base = sid * TPW16
    pltpu.sync_copy(oh1_hbm.at[pl.ds(base, TPW16), :], oh1v)
    pltpu.sync_copy(oh2_hbm.at[pl.ds(base, TPW16), :], oh2v)
    iota16 = lax.broadcasted_iota(jnp.int32, (16,), 0)
    zeros16 = jnp.zeros((16,), jnp.int32)

    def _cnt(j, c):
        c = jnp.where(iota16 == oh1v[j], c + 1, c)
        return jnp.where(iota16 == oh2v[j], c + 1, c)
    cnt = lax.fori_loop(0, TPW16, _cnt, zeros16)
    cntv[...] = cnt
    pltpu.sync_copy(cntv, sharedv.at[pl.ds(sid * 16, 16)])
    plsc.subcore_barrier()
    for w in range(16):
        pltpu.sync_copy(sharedv.at[pl.ds(w * 16, 16)], rdv.at[w])

    totals = zeros16
    mybase = zeros16
    for w in range(16):
        row = rdv[w]
        totals = totals + row
        gate = jnp.minimum(jnp.maximum(sid - w, 0), 1)
        mybase = mybase + row * gate
    aligned = ((totals + (TM - 1)) >> 8) << 8
    csum = aligned
    for dstep in (1, 2, 4, 8):
        shifted = _dyn_gather(csum, jnp.maximum(iota16 - dstep, 0))
        csum = jnp.where(iota16 >= dstep, csum + shifted, csum)
    offs = csum - aligned
    tile_starts = jnp.where(iota16 < E, offs >> 8,
                            jnp.full((16,), BIG, jnp.int32))
    nextpos = offs + mybase

    @pl.when((cid == 0) & (sid == 0))
    def _():
        toffv[...] = tile_starts
        pltpu.sync_copy(toffv, toff_hbm)

    def _mk_pass(ohv, pv):
        def _chunk(cc, np_):
            packed = zeros16
            for j in range(16):
                es = ohv[cc * 16 + j]
                s_splat = _dyn_gather(np_, es)
                packed = jnp.where(iota16 == j, s_splat, packed)
                np_ = jnp.where(iota16 == es, np_ + 1, np_)
            pv[pl.ds(cc * 16, 16)] = packed
            return np_
        return _chunk

    nextpos = lax.fori_loop(0, TPW16 // 16, _mk_pass(oh1v, pos1v), nextpos)
    nextpos = lax.fori_loop(0, TPW16 // 16, _mk_pass(oh2v, pos2v), nextpos)

    @pl.when(cid == 0)
    def _():
        pltpu.sync_copy(pos1v, pos1_hbm.at[pl.ds(base, TPW16)])
        pltpu.sync_copy(pos2v, pos2_hbm.at[pl.ds(base, TPW16)])

    for j in range(TPW16 // CH // 2):
        jj = cid * (TPW16 // CH // 2) + j
        tok = base + jj * CH
        for h in range(CH // 16):
            idxv[pl.ds(h * 16, 16)] = pos1v[pl.ds(jj * CH + h * 16, 16)]
        pltpu.sync_copy(x_hbm.at[pl.ds(tok, CH), :], xbuf)
        pltpu.async_copy(xbuf, xs_hbm.at[idxv], sem).wait()
        for h in range(CH // 16):
            idxv[pl.ds(h * 16, 16)] = pos2v[pl.ds(jj * CH + h * 16, 16)]
        pltpu.async_copy(xbuf, xs_hbm.at[idxv], sem).wait()


def _routing(oh1, oh2, x_flat):
    fn = pl.kernel(
        _routing_body,
        out_type=[
            jax.ShapeDtypeStruct((N,), jnp.int32),
            jax.ShapeDtypeStruct((N,), jnp.int32),
            jax.ShapeDtypeStruct((16,), jnp.int32),
            jax.ShapeDtypeStruct((NPAD, D), jnp.float32),
        ],
        mesh=plsc.VectorSubcoreMesh(core_axis_name="c", subcore_axis_name="s",
                                    num_cores=2, num_subcores=16),
        scratch_types=[
            pltpu.VMEM((TPW16, 16), jnp.int32),
            pltpu.VMEM((TPW16, 16), jnp.int32),
            pltpu.VMEM((TPW16,), jnp.int32),
            pltpu.VMEM((TPW16,), jnp.int32),
            pltpu.VMEM((16, 16), jnp.int32),
            pltpu.VMEM((16,), jnp.int32),
            pltpu.VMEM((16,), jnp.int32),
            pltpu.VMEM((CH, D), jnp.float32),
            pltpu.VMEM((CH,), jnp.int32),
            pltpu.VMEM_SHARED((256,), jnp.int32),
            pltpu.SemaphoreType.DMA,
        ],
    )
    return fn(oh1, oh2, x_flat)


def _ffn_body(toff, xref, w1ref, b1ref, w2ref, b2ref, yref):
    xt = xref[...]
    h = jax.nn.relu(jnp.dot(xt, w1ref[0], preferred_element_type=jnp.float32)
                    + b1ref[0])
    yref[...] = (jnp.dot(h, w2ref[0], preferred_element_type=jnp.float32)
                 + b2ref[0])


def _tile_expert(t, toff):
    e = jnp.int32(0)
    for j in range(1, 16):
        e = e + (t >= toff[j]).astype(jnp.int32)
    return e


def _ffn(toff, xs, W1, b1, W2, b2):
    grid_spec = pltpu.PrefetchScalarGridSpec(
        num_scalar_prefetch=1,
        grid=(NTILES,),
        in_specs=[
            pl.BlockSpec((TM, D), lambda t, toff: (t, 0)),
            pl.BlockSpec((1, D, F), lambda t, toff: (_tile_expert(t, toff), 0, 0)),
            pl.BlockSpec((1, 1, F), lambda t, toff: (_tile_expert(t, toff), 0, 0)),
            pl.BlockSpec((1, F, D), lambda t, toff: (_tile_expert(t, toff), 0, 0)),
            pl.BlockSpec((1, 1, D), lambda t, toff: (_tile_expert(t, toff), 0, 0)),
        ],
        out_specs=pl.BlockSpec((TM, D), lambda t, toff: (t, 0)),
    )
    return pl.pallas_call(
        _ffn_body,
        grid_spec=grid_spec,
        out_shape=jax.ShapeDtypeStruct((NPAD, D), jnp.float32),
    )(toff, xs, W1, b1.reshape(E, 1, F), W2, b2.reshape(E, 1, D))


def _combine_body(y_hbm, pos1_hbm, pos2_hbm, v1_hbm, v2_hbm, out_hbm,
                  bufa, bufb, bufo, wav, wbv, idxv, sem):
    cid = lax.axis_index("c")
    sid = lax.axis_index("s")
    wid = sid * 2 + cid
    base = wid * TPW
    for c in range(TPW // CH):
        tok = base + c * CH
        pltpu.sync_copy(pos1_hbm.at[pl.ds(tok, CH)], idxv)
        pltpu.async_copy(y_hbm.at[idxv], bufa, sem).wait()
        pltpu.sync_copy(pos2_hbm.at[pl.ds(tok, CH)], idxv)
        pltpu.async_copy(y_hbm.at[idxv], bufb, sem).wait()
        pltpu.sync_copy(v1_hbm.at[pl.ds(tok, CH), :], wav)
        pltpu.sync_copy(v2_hbm.at[pl.ds(tok, CH), :], wbv)

        def _row(r, carry):
            wa = wav[r]
            wb = wbv[r]
            for cv in range(D // 16):
                sl = pl.ds(cv * 16, 16)
                bufo[r, sl] = bufa[r, sl] * wa + bufb[r, sl] * wb
            return carry
        lax.fori_loop(0, CH, _row, 0)
        pltpu.sync_copy(bufo, out_hbm.at[pl.ds(tok, CH), :])


def _combine(y, pos1, pos2, v1b, v2b):
    fn = pl.kernel(
        _combine_body,
        out_type=jax.ShapeDtypeStruct((N, D), jnp.float32),
        mesh=plsc.VectorSubcoreMesh(core_axis_name="c", subcore_axis_name="s",
                                    num_cores=2, num_subcores=16),
        scratch_types=[
            pltpu.VMEM((CH, D), jnp.float32),
            pltpu.VMEM((CH, D), jnp.float32),
            pltpu.VMEM((CH, D), jnp.float32),
            pltpu.VMEM((CH, 16), jnp.float32),
            pltpu.VMEM((CH, 16), jnp.float32),
            pltpu.VMEM((CH,), jnp.int32),
            pltpu.SemaphoreType.DMA,
        ],
    )
    return fn(y, pos1, pos2, v1b, v2b)


def kernel(x, Wg, bg, W1, b1, W2, b2):
    B, T, _ = x.shape
    x_flat = x.reshape(N, D)
    v1b, v2b, oh1, oh2 = _router(x_flat, Wg, bg)
    pos1, pos2, toff, xs = _routing(oh1, oh2, x_flat)
    y = _ffn(toff, xs, W1, b1, W2, b2)
    out = _combine(y, pos1, pos2, v1b, v2b)
    return out.reshape(B, T, D)
